# baseline (speedup 1.0000x reference)
import functools

import jax
import jax.numpy as jnp
from jax import lax
from jax.experimental import pallas as pl
from jax.experimental.pallas import tpu as pltpu
from jax.experimental.pallas import tpu_sc as plsc

B = 4096
C = 64
HW = 256


def _make_sc_gather(n_rows, d):
    info = plsc.get_sparse_core_info()
    nc, ns = info.num_cores, info.num_subcores
    nw = nc * ns
    assert n_rows % (8 * nw) == 0
    b_per_w = n_rows // nw
    mesh = plsc.VectorSubcoreMesh(core_axis_name="c", subcore_axis_name="s")

    @functools.partial(
        pl.kernel,
        mesh=mesh,
        out_type=jax.ShapeDtypeStruct((n_rows, d), jnp.float32),
        scratch_types=[
            pltpu.VMEM((b_per_w,), jnp.int32),
            pltpu.VMEM((b_per_w, d), jnp.float32),
            pltpu.SemaphoreType.DMA,
        ],
        compiler_params=pltpu.CompilerParams(use_tc_tiling_on_sc=False),
    )
    def gather_k(idx_hbm, table_hbm, out_hbm, idx_v, rows_v, sem):
        wid = lax.axis_index("s") * nc + lax.axis_index("c")
        base = wid * b_per_w
        pltpu.sync_copy(idx_hbm.at[pl.ds(base, b_per_w)], idx_v)
        pltpu.async_copy(table_hbm.at[idx_v], rows_v, sem).wait()
        pltpu.sync_copy(rows_v, out_hbm.at[pl.ds(base, b_per_w)])

    return gather_k


def _make_tc_stream(bb, nbuf):
    n_chunks = B // bb
    outer_n = n_chunks // nbuf

    def body(shift_hbm, x_hbm, o_hbm, shift_v, *rest):
        in_bufs = rest[0:nbuf]
        out_bufs = rest[nbuf : 2 * nbuf]
        in_sems = rest[2 * nbuf : 3 * nbuf]
        out_sems = rest[3 * nbuf : 4 * nbuf]
        sem_s = rest[4 * nbuf]

        pltpu.make_async_copy(shift_hbm, shift_v, sem_s).start()

        for b in range(nbuf):
            pltpu.make_async_copy(
                x_hbm.at[pl.ds(b * bb, bb)], in_bufs[b], in_sems[b]
            ).start()

        pltpu.make_async_copy(shift_hbm, shift_v, sem_s).wait()

        def outer(o, carry):
            for b in range(nbuf):
                g = o * nbuf + b
                pltpu.make_async_copy(
                    x_hbm.at[pl.ds(g * bb, bb)], in_bufs[b], in_sems[b]
                ).wait()

                @pl.when(o > 0)
                def _wait_out():
                    pltpu.make_async_copy(
                        out_bufs[b], o_hbm.at[pl.ds(g * bb, bb)], out_sems[b]
                    ).wait()

                sh = shift_v[pl.ds(g * bb, bb), :]
                out_bufs[b][...] = in_bufs[b][...] - sh[:, :, None]
                pltpu.make_async_copy(
                    out_bufs[b], o_hbm.at[pl.ds(g * bb, bb)], out_sems[b]
                ).start()

                @pl.when(o < outer_n - 1)
                def _next_in():
                    pltpu.make_async_copy(
                        x_hbm.at[pl.ds((g + nbuf) * bb, bb)], in_bufs[b], in_sems[b]
                    ).start()

            return carry

        lax.fori_loop(0, outer_n, outer, 0)

        for b in range(nbuf):
            pltpu.make_async_copy(
                out_bufs[b], o_hbm.at[pl.ds(b * bb, bb)], out_sems[b]
            ).wait()

    return pl.pallas_call(
        body,
        in_specs=[
            pl.BlockSpec(memory_space=pltpu.HBM),
            pl.BlockSpec(memory_space=pltpu.HBM),
        ],
        out_specs=pl.BlockSpec(memory_space=pltpu.HBM),
        out_shape=jax.ShapeDtypeStruct((B, C, HW), jnp.float32),
        scratch_shapes=(
            [pltpu.VMEM((B, C), jnp.float32)]
            + [pltpu.VMEM((bb, C, HW), jnp.float32) for _ in range(2 * nbuf)]
            + [pltpu.SemaphoreType.DMA for _ in range(2 * nbuf + 1)]
        ),
    )


def kernel(x, y, log_det_jac, z, factors):
    y32 = y.astype(jnp.int32)
    shift = _make_sc_gather(B, C)(y32, factors)
    x3 = x.reshape(B, C, HW)
    out3 = _make_tc_stream(bb=32, nbuf=4)(shift, x3)
    return (out3.reshape(x.shape), log_det_jac, z)
